# baseline (speedup 1.0000x reference)
import functools

import jax
import jax.numpy as jnp
from jax import lax
from jax.experimental import pallas as pl
from jax.experimental.pallas import tpu as pltpu
from jax.experimental.pallas import tpu_sc as plsc

H = 1024
FF = 3584
E = 8
TOPK = 2
EPS = 1e-06
T = 2048
A = T * TOPK
BLK = 256
NBLK = A // BLK + E
P = NBLK * BLK
LANES = 128

NTILES = 32
TPT = T // NTILES
CH = 32


def _routing_kernel(hs_ref, rmsw_ref, gw_ref, hsn_ref, pos_ref, tw_ref,
                    blk_ref):
    x = hs_ref[...]
    var = jnp.mean(x * x, axis=1, keepdims=True)
    xn = x * lax.rsqrt(var + EPS) * rmsw_ref[...]
    hsn_ref[...] = xn

    logits = jnp.dot(xn, gw_ref[...], preferred_element_type=jnp.float32)
    lane = lax.broadcasted_iota(jnp.int32, (T, LANES), 1)
    valid = lane < E
    logits = jnp.where(valid, logits, -1e30)
    m = jnp.max(logits, axis=1, keepdims=True)
    p = jnp.where(valid, jnp.exp(logits - m), 0.0)
    sc = p / jnp.sum(p, axis=1, keepdims=True)

    m1 = jnp.max(sc, axis=1, keepdims=True)
    i1 = jnp.min(jnp.where((sc == m1) & valid, lane, LANES), axis=1,
                 keepdims=True)
    oh1 = lane == i1
    sc2 = jnp.where(oh1, -1.0, sc)
    m2 = jnp.max(sc2, axis=1, keepdims=True)
    i2 = jnp.min(jnp.where((sc2 == m2) & valid, lane, LANES), axis=1,
                 keepdims=True)
    oh2 = lane == i2
    ssum = m1 + m2
    tw_ref[...] = jnp.where(lane == 0, m1 / ssum,
                            jnp.where(lane == 1, m2 / ssum, 0.0))

    oh = (oh1 | oh2).astype(jnp.int32)
    c = oh
    d = 1
    while d < T:
        c = c + jnp.concatenate(
            [jnp.zeros((d, LANES), jnp.int32), c[:T - d]], axis=0)
        d *= 2
    cex = c - oh
    counts = c[T - 1:T, :]

    nb = jnp.where(lane[0:1, :] < E,
                   jnp.right_shift(counts + (BLK - 1), 8), 0)
    row = lax.broadcasted_iota(jnp.int32, (LANES, LANES), 0)
    col = lax.broadcasted_iota(jnp.int32, (LANES, LANES), 1)
    tri = (row < col).astype(jnp.float32)
    nb8 = jnp.broadcast_to(nb.astype(jnp.float32), (8, LANES))
    blk_base = jnp.dot(nb8, tri, preferred_element_type=jnp.float32)
    pad_base = (blk_base[0:1, :] * BLK).astype(jnp.int32)

    rank1 = jnp.sum(jnp.where(oh1, cex, 0), axis=1, keepdims=True)
    rank2 = jnp.sum(jnp.where(oh2, cex, 0), axis=1, keepdims=True)
    base1 = jnp.sum(jnp.where(oh1, pad_base, 0), axis=1, keepdims=True)
    base2 = jnp.sum(jnp.where(oh2, pad_base, 0), axis=1, keepdims=True)
    pos_ref[...] = jnp.where(lane == 0, base1 + rank1,
                             jnp.where(lane == 1, base2 + rank2, 0))

    lane_row = lane[0:1, :]
    nbi = nb.astype(jnp.int32)
    total = jnp.sum(nbi, axis=1, keepdims=True)
    owner = jnp.zeros((1, LANES), jnp.int32)
    for e in range(E):
        base_e = jnp.sum(jnp.where(lane_row == e, pad_base, 0), axis=1,
                         keepdims=True)
        owner = owner + (base_e <= lane_row * BLK).astype(jnp.int32)
    owner = owner - 1
    actv = (lane_row < total).astype(jnp.int32)
    e_last = jnp.sum(jnp.where(lane_row == total - 1, owner, 0), axis=1,
                     keepdims=True)
    blk_e = jnp.where(actv == 1, owner, e_last)
    blk_ref[...] = jnp.broadcast_to(
        jnp.where(lax.broadcasted_iota(jnp.int32, (8, LANES), 0) == 0,
                  blk_e, actv), (8, LANES))


def _routing(hs, rms_weight, gate_w_pad):
    return pl.pallas_call(
        _routing_kernel,
        out_shape=(
            jax.ShapeDtypeStruct((T, H), jnp.float32),
            jax.ShapeDtypeStruct((T, LANES), jnp.int32),
            jax.ShapeDtypeStruct((T, LANES), jnp.float32),
            jax.ShapeDtypeStruct((8, LANES), jnp.int32),
        ),
    )(hs, rms_weight, gate_w_pad)


def _sc_scatter_body(hsn, pos, xpad, rows0_v, rows1_v, p00_v, p01_v, p10_v,
                     p11_v, semr, sems):
    wid = lax.axis_index("s") * 2 + lax.axis_index("c")
    t0 = wid * TPT
    t1 = wid * TPT + CH
    r0 = pltpu.async_copy(hsn.at[pl.ds(t0, CH)], rows0_v, semr)
    r1 = pltpu.async_copy(hsn.at[pl.ds(t1, CH)], rows1_v, semr)
    pltpu.sync_copy(pos.at[0, pl.ds(t0, CH)], p00_v)
    pltpu.sync_copy(pos.at[1, pl.ds(t0, CH)], p01_v)
    pltpu.sync_copy(pos.at[0, pl.ds(t1, CH)], p10_v)
    pltpu.sync_copy(pos.at[1, pl.ds(t1, CH)], p11_v)
    r0.wait()
    s00 = pltpu.async_copy(rows0_v, xpad.at[p00_v], sems)
    s01 = pltpu.async_copy(rows0_v, xpad.at[p01_v], sems)
    r1.wait()
    s10 = pltpu.async_copy(rows1_v, xpad.at[p10_v], sems)
    s11 = pltpu.async_copy(rows1_v, xpad.at[p11_v], sems)
    s00.wait()
    s01.wait()
    s10.wait()
    s11.wait()


def _scatter_sc(hsn, pos_slots):
    mesh = plsc.VectorSubcoreMesh(core_axis_name="c", subcore_axis_name="s")
    f = functools.partial(
        pl.kernel,
        mesh=mesh,
        out_type=jax.ShapeDtypeStruct((P, H), jnp.float32),
        scratch_types=[
            pltpu.VMEM((CH, H), jnp.float32),
            pltpu.VMEM((CH, H), jnp.float32),
            pltpu.VMEM((CH,), jnp.int32),
            pltpu.VMEM((CH,), jnp.int32),
            pltpu.VMEM((CH,), jnp.int32),
            pltpu.VMEM((CH,), jnp.int32),
            pltpu.SemaphoreType.DMA,
            pltpu.SemaphoreType.DMA,
        ],
    )(_sc_scatter_body)
    return f(hsn, pos_slots)


FT = 7
FFT = FF // FT


def _ffn_kernel(be_ref, act_ref, x_ref, w1_ref, w3_ref, w2_ref, o_ref,
                acc_ref):
    f = pl.program_id(0)
    b = pl.program_id(1)

    @pl.when(act_ref[b] == 1)
    def _():
        x = x_ref[pl.ds(b * BLK, BLK), :].astype(jnp.bfloat16)
        h1 = jnp.dot(x, w1_ref[0].astype(jnp.bfloat16),
                     preferred_element_type=jnp.float32)
        h3 = jnp.dot(x, w3_ref[0].astype(jnp.bfloat16),
                     preferred_element_type=jnp.float32)
        a = ((h1 * lax.logistic(h1)) * h3).astype(jnp.bfloat16)
        part = jnp.dot(a, w2_ref[0].astype(jnp.bfloat16),
                       preferred_element_type=jnp.float32)

        @pl.when(f == 0)
        def _():
            acc_ref[pl.ds(b * BLK, BLK), :] = part.astype(jnp.bfloat16)

        @pl.when((f > 0) & (f < FT - 1))
        def _():
            prev = acc_ref[pl.ds(b * BLK, BLK), :].astype(jnp.float32)
            acc_ref[pl.ds(b * BLK, BLK), :] = (prev + part).astype(
                jnp.bfloat16)

        @pl.when(f == FT - 1)
        def _():
            prev = acc_ref[pl.ds(b * BLK, BLK), :].astype(jnp.float32)
            o_ref[...] = prev + part


def _ffn_grouped(blk_e, blk_act, x_pad, w1, w3, w2):
    return pl.pallas_call(
        _ffn_kernel,
        grid_spec=pltpu.PrefetchScalarGridSpec(
            num_scalar_prefetch=2,
            grid=(FT, NBLK),
            in_specs=[
                pl.BlockSpec((P, H), lambda f, b, be, act: (0, 0)),
                pl.BlockSpec((1, H, FFT), lambda f, b, be, act: (be[b], 0, f)),
                pl.BlockSpec((1, H, FFT), lambda f, b, be, act: (be[b], 0, f)),
                pl.BlockSpec((1, FFT, H), lambda f, b, be, act: (be[b], f, 0)),
            ],
            out_specs=pl.BlockSpec(
                (BLK, H),
                lambda f, b, be, act: (jnp.where(f == FT - 1, b, 0), 0)),
            scratch_shapes=[pltpu.VMEM((P, H), jnp.bfloat16)],
        ),
        out_shape=jax.ShapeDtypeStruct((P, H), jnp.float32),
    )(blk_e, blk_act, x_pad, w1, w3, w2)


def _sc_gather_body(outpad, pos, gat, rows0_v, rows1_v, p0_v, p1_v, semg,
                    semw):
    wid = lax.axis_index("s") * 2 + lax.axis_index("c")
    for ch in range(TPT // CH):
        tbase = wid * TPT + ch * CH
        pltpu.sync_copy(pos.at[0, pl.ds(tbase, CH)], p0_v)
        pltpu.sync_copy(pos.at[1, pl.ds(tbase, CH)], p1_v)
        g0 = pltpu.async_copy(outpad.at[p0_v], rows0_v, semg)
        g1 = pltpu.async_copy(outpad.at[p1_v], rows1_v, semg)
        g0.wait()
        w0 = pltpu.async_copy(rows0_v, gat.at[0, pl.ds(tbase, CH)], semw)
        g1.wait()
        w1 = pltpu.async_copy(rows1_v, gat.at[1, pl.ds(tbase, CH)], semw)
        w0.wait()
        w1.wait()


def _gather_sc(out_pad, pos_slots):
    mesh = plsc.VectorSubcoreMesh(core_axis_name="c", subcore_axis_name="s")
    f = functools.partial(
        pl.kernel,
        mesh=mesh,
        out_type=jax.ShapeDtypeStruct((TOPK, T, H), jnp.float32),
        scratch_types=[
            pltpu.VMEM((CH, H), jnp.float32),
            pltpu.VMEM((CH, H), jnp.float32),
            pltpu.VMEM((CH,), jnp.int32),
            pltpu.VMEM((CH,), jnp.int32),
            pltpu.SemaphoreType.DMA,
            pltpu.SemaphoreType.DMA,
        ],
    )(_sc_gather_body)
    return f(out_pad, pos_slots)


def _combine_kernel(inp_ref, g_ref, tw_ref, o_ref):
    tw0 = tw_ref[:, 0:1]
    tw1 = tw_ref[:, 1:2]
    o_ref[...] = inp_ref[...] + tw0 * g_ref[0] + tw1 * g_ref[1]


def _combine(inp, gathered, tw):
    nb = 8
    tb = T // nb
    return pl.pallas_call(
        _combine_kernel,
        grid=(nb,),
        in_specs=[
            pl.BlockSpec((tb, H), lambda i: (i, 0)),
            pl.BlockSpec((TOPK, tb, H), lambda i: (0, i, 0)),
            pl.BlockSpec((tb, LANES), lambda i: (i, 0)),
        ],
        out_specs=pl.BlockSpec((tb, H), lambda i: (i, 0)),
        out_shape=jax.ShapeDtypeStruct((T, H), jnp.float32),
    )(inp, gathered, tw)


def kernel(hidden_states, rms_weight, gate_w, w1, w2, w3):
    b, s, h = hidden_states.shape
    hs = hidden_states.reshape(T, H)
    rmsw = rms_weight.reshape(1, H)
    gwp = jnp.pad(gate_w, ((0, 0), (0, LANES - E)))

    hsn, pos128, tw128, blk = _routing(hs, rmsw, gwp)
    pos_slots = jnp.transpose(pos128[:, :TOPK]).astype(jnp.int32)
    blk_e = blk[0, :NBLK]
    act = blk[1, :NBLK]

    x_pad = _scatter_sc(hsn, pos_slots)
    out_pad = _ffn_grouped(blk_e, act, x_pad, w1, w3, w2)
    gathered = _gather_sc(out_pad, pos_slots)
    final = _combine(hs, gathered, tw128)
    return final.reshape(b, s, h)


# speedup vs baseline: 1.1534x; 1.1534x over previous
import functools

import jax
import jax.numpy as jnp
from jax import lax
from jax.experimental import pallas as pl
from jax.experimental.pallas import tpu as pltpu
from jax.experimental.pallas import tpu_sc as plsc

H = 1024
FF = 3584
E = 8
TOPK = 2
EPS = 1e-06
T = 2048
A = T * TOPK
BLK = 1024
BLKSH = 10
NBLK = A // BLK + E
P = NBLK * BLK
LANES = 128

NTILES = 32
TPT = T // NTILES
CH = 32


def _routing_kernel(hs_ref, rmsw_ref, gw_ref, hsn_ref, pos_ref, tw_ref,
                    blk_ref):
    x = hs_ref[...]
    var = jnp.mean(x * x, axis=1, keepdims=True)
    xn = x * lax.rsqrt(var + EPS) * rmsw_ref[...]
    hsn_ref[...] = xn

    logits = jnp.dot(xn, gw_ref[...], preferred_element_type=jnp.float32)
    lane = lax.broadcasted_iota(jnp.int32, (T, LANES), 1)
    valid = lane < E
    logits = jnp.where(valid, logits, -1e30)
    m = jnp.max(logits, axis=1, keepdims=True)
    p = jnp.where(valid, jnp.exp(logits - m), 0.0)
    sc = p / jnp.sum(p, axis=1, keepdims=True)

    m1 = jnp.max(sc, axis=1, keepdims=True)
    i1 = jnp.min(jnp.where((sc == m1) & valid, lane, LANES), axis=1,
                 keepdims=True)
    oh1 = lane == i1
    sc2 = jnp.where(oh1, -1.0, sc)
    m2 = jnp.max(sc2, axis=1, keepdims=True)
    i2 = jnp.min(jnp.where((sc2 == m2) & valid, lane, LANES), axis=1,
                 keepdims=True)
    oh2 = lane == i2
    ssum = m1 + m2
    tw_ref[...] = jnp.where(lane == 0, m1 / ssum,
                            jnp.where(lane == 1, m2 / ssum, 0.0))

    oh = (oh1 | oh2).astype(jnp.int32)
    c = oh
    d = 1
    while d < T:
        c = c + jnp.concatenate(
            [jnp.zeros((d, LANES), jnp.int32), c[:T - d]], axis=0)
        d *= 2
    cex = c - oh
    counts = c[T - 1:T, :]

    nb = jnp.where(lane[0:1, :] < E,
                   jnp.right_shift(counts + (BLK - 1), BLKSH), 0)
    row = lax.broadcasted_iota(jnp.int32, (LANES, LANES), 0)
    col = lax.broadcasted_iota(jnp.int32, (LANES, LANES), 1)
    tri = (row < col).astype(jnp.float32)
    nb8 = jnp.broadcast_to(nb.astype(jnp.float32), (8, LANES))
    blk_base = jnp.dot(nb8, tri, preferred_element_type=jnp.float32)
    pad_base = (blk_base[0:1, :] * BLK).astype(jnp.int32)

    rank1 = jnp.sum(jnp.where(oh1, cex, 0), axis=1, keepdims=True)
    rank2 = jnp.sum(jnp.where(oh2, cex, 0), axis=1, keepdims=True)
    base1 = jnp.sum(jnp.where(oh1, pad_base, 0), axis=1, keepdims=True)
    base2 = jnp.sum(jnp.where(oh2, pad_base, 0), axis=1, keepdims=True)
    pos_ref[...] = jnp.where(lane == 0, base1 + rank1,
                             jnp.where(lane == 1, base2 + rank2, 0))

    lane_row = lane[0:1, :]
    nbi = nb.astype(jnp.int32)
    total = jnp.sum(nbi, axis=1, keepdims=True)
    owner = jnp.zeros((1, LANES), jnp.int32)
    for e in range(E):
        base_e = jnp.sum(jnp.where(lane_row == e, pad_base, 0), axis=1,
                         keepdims=True)
        owner = owner + (base_e <= lane_row * BLK).astype(jnp.int32)
    owner = owner - 1
    actv = (lane_row < total).astype(jnp.int32)
    e_last = jnp.sum(jnp.where(lane_row == total - 1, owner, 0), axis=1,
                     keepdims=True)
    blk_e = jnp.where(actv == 1, owner, e_last)
    blk_ref[...] = jnp.broadcast_to(
        jnp.where(lax.broadcasted_iota(jnp.int32, (8, LANES), 0) == 0,
                  blk_e, actv), (8, LANES))


def _routing(hs, rms_weight, gate_w_pad):
    return pl.pallas_call(
        _routing_kernel,
        out_shape=(
            jax.ShapeDtypeStruct((T, H), jnp.float32),
            jax.ShapeDtypeStruct((T, LANES), jnp.int32),
            jax.ShapeDtypeStruct((T, LANES), jnp.float32),
            jax.ShapeDtypeStruct((8, LANES), jnp.int32),
        ),
    )(hs, rms_weight, gate_w_pad)


def _sc_scatter_body(hsn, pos, xpad, rows0_v, rows1_v, p00_v, p01_v, p10_v,
                     p11_v, semr, sems):
    wid = lax.axis_index("s") * 2 + lax.axis_index("c")
    t0 = wid * TPT
    t1 = wid * TPT + CH
    r0 = pltpu.async_copy(hsn.at[pl.ds(t0, CH)], rows0_v, semr)
    r1 = pltpu.async_copy(hsn.at[pl.ds(t1, CH)], rows1_v, semr)
    pltpu.sync_copy(pos.at[0, pl.ds(t0, CH)], p00_v)
    pltpu.sync_copy(pos.at[1, pl.ds(t0, CH)], p01_v)
    pltpu.sync_copy(pos.at[0, pl.ds(t1, CH)], p10_v)
    pltpu.sync_copy(pos.at[1, pl.ds(t1, CH)], p11_v)
    r0.wait()
    s00 = pltpu.async_copy(rows0_v, xpad.at[p00_v], sems)
    s01 = pltpu.async_copy(rows0_v, xpad.at[p01_v], sems)
    r1.wait()
    s10 = pltpu.async_copy(rows1_v, xpad.at[p10_v], sems)
    s11 = pltpu.async_copy(rows1_v, xpad.at[p11_v], sems)
    s00.wait()
    s01.wait()
    s10.wait()
    s11.wait()


def _scatter_sc(hsn, pos_slots):
    mesh = plsc.VectorSubcoreMesh(core_axis_name="c", subcore_axis_name="s")
    f = functools.partial(
        pl.kernel,
        mesh=mesh,
        out_type=jax.ShapeDtypeStruct((P, H), jnp.float32),
        scratch_types=[
            pltpu.VMEM((CH, H), jnp.float32),
            pltpu.VMEM((CH, H), jnp.float32),
            pltpu.VMEM((CH,), jnp.int32),
            pltpu.VMEM((CH,), jnp.int32),
            pltpu.VMEM((CH,), jnp.int32),
            pltpu.VMEM((CH,), jnp.int32),
            pltpu.SemaphoreType.DMA,
            pltpu.SemaphoreType.DMA,
        ],
    )(_sc_scatter_body)
    return f(hsn, pos_slots)


FT = 7
FFT = FF // FT


def _ffn_kernel(be_ref, act_ref, x_ref, w1_ref, w3_ref, w2_ref, o_ref,
                xb_ref, acc_ref):
    b = pl.program_id(0)
    f = pl.program_id(1)

    @pl.when(act_ref[b] == 1)
    def _():
        @pl.when(f == 0)
        def _():
            xb_ref[...] = x_ref[...].astype(jnp.bfloat16)

        xb = xb_ref[...]
        h1 = jnp.dot(xb, w1_ref[0].astype(jnp.bfloat16),
                     preferred_element_type=jnp.float32)
        h3 = jnp.dot(xb, w3_ref[0].astype(jnp.bfloat16),
                     preferred_element_type=jnp.float32)
        a = ((h1 * lax.logistic(h1)) * h3).astype(jnp.bfloat16)
        part = jnp.dot(a, w2_ref[0].astype(jnp.bfloat16),
                       preferred_element_type=jnp.float32)

        @pl.when(f == 0)
        def _():
            acc_ref[...] = part

        @pl.when((f > 0) & (f < FT - 1))
        def _():
            acc_ref[...] = acc_ref[...] + part

        @pl.when(f == FT - 1)
        def _():
            o_ref[...] = acc_ref[...] + part


def _ffn_grouped(blk_e, blk_act, x_pad, w1, w3, w2):
    def feff(b, f, act):
        return jnp.where(act[b] == 1, f, FT - 1)

    return pl.pallas_call(
        _ffn_kernel,
        grid_spec=pltpu.PrefetchScalarGridSpec(
            num_scalar_prefetch=2,
            grid=(NBLK, FT),
            in_specs=[
                pl.BlockSpec(
                    (BLK, H),
                    lambda b, f, be, act: (jnp.where(act[b] == 1, b, 0), 0)),
                pl.BlockSpec(
                    (1, H, FFT),
                    lambda b, f, be, act: (be[b], 0, feff(b, f, act))),
                pl.BlockSpec(
                    (1, H, FFT),
                    lambda b, f, be, act: (be[b], 0, feff(b, f, act))),
                pl.BlockSpec(
                    (1, FFT, H),
                    lambda b, f, be, act: (be[b], feff(b, f, act), 0)),
            ],
            out_specs=pl.BlockSpec((BLK, H), lambda b, f, be, act: (b, 0)),
            scratch_shapes=[
                pltpu.VMEM((BLK, H), jnp.bfloat16),
                pltpu.VMEM((BLK, H), jnp.float32),
            ],
        ),
        out_shape=jax.ShapeDtypeStruct((P, H), jnp.float32),
    )(blk_e, blk_act, x_pad, w1, w3, w2)


def _sc_gather_body(outpad, pos, gat, rows0_v, rows1_v, p0_v, p1_v, semg,
                    semw):
    wid = lax.axis_index("s") * 2 + lax.axis_index("c")
    for ch in range(TPT // CH):
        tbase = wid * TPT + ch * CH
        pltpu.sync_copy(pos.at[0, pl.ds(tbase, CH)], p0_v)
        pltpu.sync_copy(pos.at[1, pl.ds(tbase, CH)], p1_v)
        g0 = pltpu.async_copy(outpad.at[p0_v], rows0_v, semg)
        g1 = pltpu.async_copy(outpad.at[p1_v], rows1_v, semg)
        g0.wait()
        w0 = pltpu.async_copy(rows0_v, gat.at[0, pl.ds(tbase, CH)], semw)
        g1.wait()
        w1 = pltpu.async_copy(rows1_v, gat.at[1, pl.ds(tbase, CH)], semw)
        w0.wait()
        w1.wait()


def _gather_sc(out_pad, pos_slots):
    mesh = plsc.VectorSubcoreMesh(core_axis_name="c", subcore_axis_name="s")
    f = functools.partial(
        pl.kernel,
        mesh=mesh,
        out_type=jax.ShapeDtypeStruct((TOPK, T, H), jnp.float32),
        scratch_types=[
            pltpu.VMEM((CH, H), jnp.float32),
            pltpu.VMEM((CH, H), jnp.float32),
            pltpu.VMEM((CH,), jnp.int32),
            pltpu.VMEM((CH,), jnp.int32),
            pltpu.SemaphoreType.DMA,
            pltpu.SemaphoreType.DMA,
        ],
    )(_sc_gather_body)
    return f(out_pad, pos_slots)


def _combine_kernel(inp_ref, g_ref, tw_ref, o_ref):
    tw0 = tw_ref[:, 0:1]
    tw1 = tw_ref[:, 1:2]
    o_ref[...] = inp_ref[...] + tw0 * g_ref[0] + tw1 * g_ref[1]


def _combine(inp, gathered, tw):
    nb = 8
    tb = T // nb
    return pl.pallas_call(
        _combine_kernel,
        grid=(nb,),
        in_specs=[
            pl.BlockSpec((tb, H), lambda i: (i, 0)),
            pl.BlockSpec((TOPK, tb, H), lambda i: (0, i, 0)),
            pl.BlockSpec((tb, LANES), lambda i: (i, 0)),
        ],
        out_specs=pl.BlockSpec((tb, H), lambda i: (i, 0)),
        out_shape=jax.ShapeDtypeStruct((T, H), jnp.float32),
    )(inp, gathered, tw)


def kernel(hidden_states, rms_weight, gate_w, w1, w2, w3):
    b, s, h = hidden_states.shape
    hs = hidden_states.reshape(T, H)
    rmsw = rms_weight.reshape(1, H)
    gwp = jnp.pad(gate_w, ((0, 0), (0, LANES - E)))

    hsn, pos128, tw128, blk = _routing(hs, rmsw, gwp)
    pos_slots = jnp.transpose(pos128[:, :TOPK]).astype(jnp.int32)
    blk_e = blk[0, :NBLK]
    act = blk[1, :NBLK]

    x_pad = _scatter_sc(hsn, pos_slots)
    out_pad = _ffn_grouped(blk_e, act, x_pad, w1, w3, w2)
    gathered = _gather_sc(out_pad, pos_slots)
    final = _combine(hs, gathered, tw128)
    return final.reshape(b, s, h)


# speedup vs baseline: 1.1907x; 1.0324x over previous
import functools

import jax
import jax.numpy as jnp
from jax import lax
from jax.experimental import pallas as pl
from jax.experimental.pallas import tpu as pltpu
from jax.experimental.pallas import tpu_sc as plsc

H = 1024
FF = 3584
E = 8
TOPK = 2
EPS = 1e-06
T = 2048
A = T * TOPK
BLK = 1024
BLKSH = 10
NBLK = A // BLK + E
P = NBLK * BLK
LANES = 128

NTILES = 32
TPT = T // NTILES
CH = 32
HW = H // 2


def _pack_bf16(lo_f32, hi_f32):
    lo = lax.bitcast_convert_type(
        lo_f32.astype(jnp.bfloat16).astype(jnp.float32), jnp.int32)
    hi = lax.bitcast_convert_type(
        hi_f32.astype(jnp.bfloat16).astype(jnp.float32), jnp.int32)
    return (hi & jnp.int32(-65536)) | lax.shift_right_logical(lo, 16)


def _unpack_bf16(p_i32):
    lo = lax.bitcast_convert_type(lax.shift_left(p_i32, 16), jnp.float32)
    hi = lax.bitcast_convert_type(p_i32 & jnp.int32(-65536), jnp.float32)
    return lo, hi


def _routing_kernel(hs_ref, rmsw_ref, gw_ref, hsn_ref, pos_ref, tw_ref,
                    blk_ref):
    x = hs_ref[...]
    var = jnp.mean(x * x, axis=1, keepdims=True)
    xn = x * lax.rsqrt(var + EPS) * rmsw_ref[...]
    hsn_ref[...] = _pack_bf16(xn[:, :HW], xn[:, HW:])

    logits = jnp.dot(xn, gw_ref[...], preferred_element_type=jnp.float32)
    lane = lax.broadcasted_iota(jnp.int32, (T, LANES), 1)
    valid = lane < E
    logits = jnp.where(valid, logits, -1e30)
    m = jnp.max(logits, axis=1, keepdims=True)
    p = jnp.where(valid, jnp.exp(logits - m), 0.0)
    sc = p / jnp.sum(p, axis=1, keepdims=True)

    m1 = jnp.max(sc, axis=1, keepdims=True)
    i1 = jnp.min(jnp.where((sc == m1) & valid, lane, LANES), axis=1,
                 keepdims=True)
    oh1 = lane == i1
    sc2 = jnp.where(oh1, -1.0, sc)
    m2 = jnp.max(sc2, axis=1, keepdims=True)
    i2 = jnp.min(jnp.where((sc2 == m2) & valid, lane, LANES), axis=1,
                 keepdims=True)
    oh2 = lane == i2
    ssum = m1 + m2
    tw_ref[...] = jnp.where(lane == 0, m1 / ssum,
                            jnp.where(lane == 1, m2 / ssum, 0.0))

    oh = (oh1 | oh2).astype(jnp.int32)
    c = oh
    d = 1
    while d < T:
        c = c + jnp.concatenate(
            [jnp.zeros((d, LANES), jnp.int32), c[:T - d]], axis=0)
        d *= 2
    cex = c - oh
    counts = c[T - 1:T, :]

    nb = jnp.where(lane[0:1, :] < E,
                   jnp.right_shift(counts + (BLK - 1), BLKSH), 0)
    row = lax.broadcasted_iota(jnp.int32, (LANES, LANES), 0)
    col = lax.broadcasted_iota(jnp.int32, (LANES, LANES), 1)
    tri = (row < col).astype(jnp.float32)
    nb8 = jnp.broadcast_to(nb.astype(jnp.float32), (8, LANES))
    blk_base = jnp.dot(nb8, tri, preferred_element_type=jnp.float32)
    pad_base = (blk_base[0:1, :] * BLK).astype(jnp.int32)

    rank1 = jnp.sum(jnp.where(oh1, cex, 0), axis=1, keepdims=True)
    rank2 = jnp.sum(jnp.where(oh2, cex, 0), axis=1, keepdims=True)
    base1 = jnp.sum(jnp.where(oh1, pad_base, 0), axis=1, keepdims=True)
    base2 = jnp.sum(jnp.where(oh2, pad_base, 0), axis=1, keepdims=True)
    pos_ref[...] = jnp.where(lane == 0, base1 + rank1,
                             jnp.where(lane == 1, base2 + rank2, 0))

    lane_row = lane[0:1, :]
    nbi = nb.astype(jnp.int32)
    total = jnp.sum(nbi, axis=1, keepdims=True)
    owner = jnp.zeros((1, LANES), jnp.int32)
    for e in range(E):
        base_e = jnp.sum(jnp.where(lane_row == e, pad_base, 0), axis=1,
                         keepdims=True)
        owner = owner + (base_e <= lane_row * BLK).astype(jnp.int32)
    owner = owner - 1
    actv = (lane_row < total).astype(jnp.int32)
    e_last = jnp.sum(jnp.where(lane_row == total - 1, owner, 0), axis=1,
                     keepdims=True)
    blk_e = jnp.where(actv == 1, owner, e_last)
    blk_ref[...] = jnp.broadcast_to(
        jnp.where(lax.broadcasted_iota(jnp.int32, (8, LANES), 0) == 0,
                  blk_e, actv), (8, LANES))


def _routing(hs, rms_weight, gate_w_pad):
    return pl.pallas_call(
        _routing_kernel,
        out_shape=(
            jax.ShapeDtypeStruct((T, HW), jnp.int32),
            jax.ShapeDtypeStruct((T, LANES), jnp.int32),
            jax.ShapeDtypeStruct((T, LANES), jnp.float32),
            jax.ShapeDtypeStruct((8, LANES), jnp.int32),
        ),
    )(hs, rms_weight, gate_w_pad)


def _sc_scatter_body(hsn, pos, xpad, rows0_v, rows1_v, p00_v, p01_v, p10_v,
                     p11_v, semr, sems):
    wid = lax.axis_index("s") * 2 + lax.axis_index("c")
    t0 = wid * TPT
    t1 = wid * TPT + CH
    r0 = pltpu.async_copy(hsn.at[pl.ds(t0, CH)], rows0_v, semr)
    r1 = pltpu.async_copy(hsn.at[pl.ds(t1, CH)], rows1_v, semr)
    pltpu.sync_copy(pos.at[0, pl.ds(t0, CH)], p00_v)
    pltpu.sync_copy(pos.at[1, pl.ds(t0, CH)], p01_v)
    pltpu.sync_copy(pos.at[0, pl.ds(t1, CH)], p10_v)
    pltpu.sync_copy(pos.at[1, pl.ds(t1, CH)], p11_v)
    r0.wait()
    s00 = pltpu.async_copy(rows0_v, xpad.at[p00_v], sems)
    s01 = pltpu.async_copy(rows0_v, xpad.at[p01_v], sems)
    r1.wait()
    s10 = pltpu.async_copy(rows1_v, xpad.at[p10_v], sems)
    s11 = pltpu.async_copy(rows1_v, xpad.at[p11_v], sems)
    s00.wait()
    s01.wait()
    s10.wait()
    s11.wait()


def _scatter_sc(hsn, pos_slots):
    mesh = plsc.VectorSubcoreMesh(core_axis_name="c", subcore_axis_name="s")
    f = functools.partial(
        pl.kernel,
        mesh=mesh,
        out_type=jax.ShapeDtypeStruct((P, HW), jnp.int32),
        scratch_types=[
            pltpu.VMEM((CH, HW), jnp.int32),
            pltpu.VMEM((CH, HW), jnp.int32),
            pltpu.VMEM((CH,), jnp.int32),
            pltpu.VMEM((CH,), jnp.int32),
            pltpu.VMEM((CH,), jnp.int32),
            pltpu.VMEM((CH,), jnp.int32),
            pltpu.SemaphoreType.DMA,
            pltpu.SemaphoreType.DMA,
        ],
    )(_sc_scatter_body)
    return f(hsn, pos_slots)


FT = 7
FFT = FF // FT


def _ffn_kernel(be_ref, act_ref, x_ref, w1_ref, w3_ref, w2_ref, o_ref,
                xb_ref, acc_ref):
    b = pl.program_id(0)
    f = pl.program_id(1)

    @pl.when(act_ref[b] == 1)
    def _():
        @pl.when(f == 0)
        def _():
            lo, hi = _unpack_bf16(x_ref[...])
            xb_ref[...] = jnp.concatenate(
                [lo, hi], axis=1).astype(jnp.bfloat16)

        xb = xb_ref[...]
        h1 = jnp.dot(xb, w1_ref[0].astype(jnp.bfloat16),
                     preferred_element_type=jnp.float32)
        h3 = jnp.dot(xb, w3_ref[0].astype(jnp.bfloat16),
                     preferred_element_type=jnp.float32)
        a = ((h1 * lax.logistic(h1)) * h3).astype(jnp.bfloat16)
        part = jnp.dot(a, w2_ref[0].astype(jnp.bfloat16),
                       preferred_element_type=jnp.float32)

        @pl.when(f == 0)
        def _():
            acc_ref[...] = part

        @pl.when((f > 0) & (f < FT - 1))
        def _():
            acc_ref[...] = acc_ref[...] + part

        @pl.when(f == FT - 1)
        def _():
            out = acc_ref[...] + part
            o_ref[...] = _pack_bf16(out[:, :HW], out[:, HW:])


def _ffn_grouped(blk_e, blk_act, x_pad, w1, w3, w2):
    def feff(b, f, act):
        return jnp.where(act[b] == 1, f, FT - 1)

    return pl.pallas_call(
        _ffn_kernel,
        grid_spec=pltpu.PrefetchScalarGridSpec(
            num_scalar_prefetch=2,
            grid=(NBLK, FT),
            in_specs=[
                pl.BlockSpec(
                    (BLK, HW),
                    lambda b, f, be, act: (jnp.where(act[b] == 1, b, 0), 0)),
                pl.BlockSpec(
                    (1, H, FFT),
                    lambda b, f, be, act: (be[b], 0, feff(b, f, act))),
                pl.BlockSpec(
                    (1, H, FFT),
                    lambda b, f, be, act: (be[b], 0, feff(b, f, act))),
                pl.BlockSpec(
                    (1, FFT, H),
                    lambda b, f, be, act: (be[b], feff(b, f, act), 0)),
            ],
            out_specs=pl.BlockSpec((BLK, HW), lambda b, f, be, act: (b, 0)),
            scratch_shapes=[
                pltpu.VMEM((BLK, H), jnp.bfloat16),
                pltpu.VMEM((BLK, H), jnp.float32),
            ],
        ),
        out_shape=jax.ShapeDtypeStruct((P, HW), jnp.int32),
    )(blk_e, blk_act, x_pad, w1, w3, w2)


def _sc_gather_body(outpad, pos, gat, rows0_v, rows1_v, p0_v, p1_v, semg,
                    semw):
    wid = lax.axis_index("s") * 2 + lax.axis_index("c")
    for ch in range(TPT // CH):
        tbase = wid * TPT + ch * CH
        pltpu.sync_copy(pos.at[0, pl.ds(tbase, CH)], p0_v)
        pltpu.sync_copy(pos.at[1, pl.ds(tbase, CH)], p1_v)
        g0 = pltpu.async_copy(outpad.at[p0_v], rows0_v, semg)
        g1 = pltpu.async_copy(outpad.at[p1_v], rows1_v, semg)
        g0.wait()
        w0 = pltpu.async_copy(rows0_v, gat.at[0, pl.ds(tbase, CH)], semw)
        g1.wait()
        w1 = pltpu.async_copy(rows1_v, gat.at[1, pl.ds(tbase, CH)], semw)
        w0.wait()
        w1.wait()


def _gather_sc(out_pad, pos_slots):
    mesh = plsc.VectorSubcoreMesh(core_axis_name="c", subcore_axis_name="s")
    f = functools.partial(
        pl.kernel,
        mesh=mesh,
        out_type=jax.ShapeDtypeStruct((TOPK, T, HW), jnp.int32),
        scratch_types=[
            pltpu.VMEM((CH, HW), jnp.int32),
            pltpu.VMEM((CH, HW), jnp.int32),
            pltpu.VMEM((CH,), jnp.int32),
            pltpu.VMEM((CH,), jnp.int32),
            pltpu.SemaphoreType.DMA,
            pltpu.SemaphoreType.DMA,
        ],
    )(_sc_gather_body)
    return f(out_pad, pos_slots)


def _combine_kernel(inp_ref, g_ref, tw_ref, o_ref):
    tw0 = tw_ref[:, 0:1]
    tw1 = tw_ref[:, 1:2]
    lo0, hi0 = _unpack_bf16(g_ref[0])
    lo1, hi1 = _unpack_bf16(g_ref[1])
    g0 = jnp.concatenate([lo0, hi0], axis=1)
    g1 = jnp.concatenate([lo1, hi1], axis=1)
    o_ref[...] = inp_ref[...] + tw0 * g0 + tw1 * g1


def _combine(inp, gathered, tw):
    nb = 8
    tb = T // nb
    return pl.pallas_call(
        _combine_kernel,
        grid=(nb,),
        in_specs=[
            pl.BlockSpec((tb, H), lambda i: (i, 0)),
            pl.BlockSpec((TOPK, tb, HW), lambda i: (0, i, 0)),
            pl.BlockSpec((tb, LANES), lambda i: (i, 0)),
        ],
        out_specs=pl.BlockSpec((tb, H), lambda i: (i, 0)),
        out_shape=jax.ShapeDtypeStruct((T, H), jnp.float32),
    )(inp, gathered, tw)


def kernel(hidden_states, rms_weight, gate_w, w1, w2, w3):
    b, s, h = hidden_states.shape
    hs = hidden_states.reshape(T, H)
    rmsw = rms_weight.reshape(1, H)
    gwp = jnp.pad(gate_w, ((0, 0), (0, LANES - E)))

    hsn, pos128, tw128, blk = _routing(hs, rmsw, gwp)
    pos_slots = jnp.transpose(pos128[:, :TOPK]).astype(jnp.int32)
    blk_e = blk[0, :NBLK]
    act = blk[1, :NBLK]

    x_pad = _scatter_sc(hsn, pos_slots)
    out_pad = _ffn_grouped(blk_e, act, x_pad, w1, w3, w2)
    gathered = _gather_sc(out_pad, pos_slots)
    final = _combine(hs, gathered, tw128)
    return final.reshape(b, s, h)


# speedup vs baseline: 1.2102x; 1.0163x over previous
import functools

import jax
import jax.numpy as jnp
from jax import lax
from jax.experimental import pallas as pl
from jax.experimental.pallas import tpu as pltpu
from jax.experimental.pallas import tpu_sc as plsc

H = 1024
FF = 3584
E = 8
TOPK = 2
EPS = 1e-06
T = 2048
A = T * TOPK
BLK = 1024
BLKSH = 10
NBLK = A // BLK + E
P = NBLK * BLK
LANES = 128

NTILES = 32
TPT = T // NTILES
CH = 64
HW = H // 2


def _pack_bf16(lo_f32, hi_f32):
    lo = lax.bitcast_convert_type(
        lo_f32.astype(jnp.bfloat16).astype(jnp.float32), jnp.int32)
    hi = lax.bitcast_convert_type(
        hi_f32.astype(jnp.bfloat16).astype(jnp.float32), jnp.int32)
    return (hi & jnp.int32(-65536)) | lax.shift_right_logical(lo, 16)


def _unpack_bf16(p_i32):
    lo = lax.bitcast_convert_type(lax.shift_left(p_i32, 16), jnp.float32)
    hi = lax.bitcast_convert_type(p_i32 & jnp.int32(-65536), jnp.float32)
    return lo, hi


def _routing_kernel(hs_ref, rmsw_ref, gw_ref, hsn_ref, pos_ref, tw_ref,
                    blk_ref):
    x = hs_ref[...]
    var = jnp.mean(x * x, axis=1, keepdims=True)
    xn = x * lax.rsqrt(var + EPS) * rmsw_ref[...]
    hsn_ref[...] = _pack_bf16(xn[:, :HW], xn[:, HW:])

    logits = jnp.dot(xn, gw_ref[...], preferred_element_type=jnp.float32)
    lane = lax.broadcasted_iota(jnp.int32, (T, LANES), 1)
    valid = lane < E
    logits = jnp.where(valid, logits, -1e30)
    m = jnp.max(logits, axis=1, keepdims=True)
    p = jnp.where(valid, jnp.exp(logits - m), 0.0)
    sc = p / jnp.sum(p, axis=1, keepdims=True)

    m1 = jnp.max(sc, axis=1, keepdims=True)
    i1 = jnp.min(jnp.where((sc == m1) & valid, lane, LANES), axis=1,
                 keepdims=True)
    oh1 = lane == i1
    sc2 = jnp.where(oh1, -1.0, sc)
    m2 = jnp.max(sc2, axis=1, keepdims=True)
    i2 = jnp.min(jnp.where((sc2 == m2) & valid, lane, LANES), axis=1,
                 keepdims=True)
    oh2 = lane == i2
    ssum = m1 + m2
    tw_ref[...] = jnp.where(lane == 0, m1 / ssum,
                            jnp.where(lane == 1, m2 / ssum, 0.0))

    oh = (oh1 | oh2).astype(jnp.int32)
    c = oh
    d = 1
    while d < T:
        c = c + jnp.concatenate(
            [jnp.zeros((d, LANES), jnp.int32), c[:T - d]], axis=0)
        d *= 2
    cex = c - oh
    counts = c[T - 1:T, :]

    nb = jnp.where(lane[0:1, :] < E,
                   jnp.right_shift(counts + (BLK - 1), BLKSH), 0)
    row = lax.broadcasted_iota(jnp.int32, (LANES, LANES), 0)
    col = lax.broadcasted_iota(jnp.int32, (LANES, LANES), 1)
    tri = (row < col).astype(jnp.float32)
    nb8 = jnp.broadcast_to(nb.astype(jnp.float32), (8, LANES))
    blk_base = jnp.dot(nb8, tri, preferred_element_type=jnp.float32)
    pad_base = (blk_base[0:1, :] * BLK).astype(jnp.int32)

    rank1 = jnp.sum(jnp.where(oh1, cex, 0), axis=1, keepdims=True)
    rank2 = jnp.sum(jnp.where(oh2, cex, 0), axis=1, keepdims=True)
    base1 = jnp.sum(jnp.where(oh1, pad_base, 0), axis=1, keepdims=True)
    base2 = jnp.sum(jnp.where(oh2, pad_base, 0), axis=1, keepdims=True)
    pos_ref[...] = jnp.where(lane == 0, base1 + rank1,
                             jnp.where(lane == 1, base2 + rank2, 0))

    lane_row = lane[0:1, :]
    nbi = nb.astype(jnp.int32)
    total = jnp.sum(nbi, axis=1, keepdims=True)
    owner = jnp.zeros((1, LANES), jnp.int32)
    for e in range(E):
        base_e = jnp.sum(jnp.where(lane_row == e, pad_base, 0), axis=1,
                         keepdims=True)
        owner = owner + (base_e <= lane_row * BLK).astype(jnp.int32)
    owner = owner - 1
    actv = (lane_row < total).astype(jnp.int32)
    e_last = jnp.sum(jnp.where(lane_row == total - 1, owner, 0), axis=1,
                     keepdims=True)
    blk_e = jnp.where(actv == 1, owner, e_last)
    obk = jnp.where(actv == 1, lane_row, total - 1)
    rows8 = lax.broadcasted_iota(jnp.int32, (8, LANES), 0)
    blk_ref[...] = jnp.where(rows8 == 0, blk_e,
                             jnp.where(rows8 == 1, actv, obk))


def _routing(hs, rms_weight, gate_w_pad):
    return pl.pallas_call(
        _routing_kernel,
        out_shape=(
            jax.ShapeDtypeStruct((T, HW), jnp.int32),
            jax.ShapeDtypeStruct((T, LANES), jnp.int32),
            jax.ShapeDtypeStruct((T, LANES), jnp.float32),
            jax.ShapeDtypeStruct((8, LANES), jnp.int32),
        ),
    )(hs, rms_weight, gate_w_pad)


def _sc_scatter_body(hsn, pos, xpad, rows_v, p0_v, p1_v, semr, sems):
    wid = lax.axis_index("s") * 2 + lax.axis_index("c")
    t0 = wid * TPT
    r0 = pltpu.async_copy(hsn.at[pl.ds(t0, CH)], rows_v, semr)
    pltpu.sync_copy(pos.at[0, pl.ds(t0, CH)], p0_v)
    pltpu.sync_copy(pos.at[1, pl.ds(t0, CH)], p1_v)
    r0.wait()
    s0 = pltpu.async_copy(rows_v, xpad.at[p0_v], sems)
    s1 = pltpu.async_copy(rows_v, xpad.at[p1_v], sems)
    s0.wait()
    s1.wait()


def _scatter_sc(hsn, pos_slots):
    mesh = plsc.VectorSubcoreMesh(core_axis_name="c", subcore_axis_name="s")
    f = functools.partial(
        pl.kernel,
        mesh=mesh,
        out_type=jax.ShapeDtypeStruct((P, HW), jnp.int32),
        scratch_types=[
            pltpu.VMEM((CH, HW), jnp.int32),
            pltpu.VMEM((CH,), jnp.int32),
            pltpu.VMEM((CH,), jnp.int32),
            pltpu.SemaphoreType.DMA,
            pltpu.SemaphoreType.DMA,
        ],
    )(_sc_scatter_body)
    return f(hsn, pos_slots)


FT = 7
FFT = FF // FT


def _ffn_kernel(be_ref, act_ref, ob_ref, x_ref, w1_ref, w3_ref, w2_ref,
                o_ref, xb_ref, acc_ref):
    b = pl.program_id(0)
    f = pl.program_id(1)

    @pl.when(act_ref[b] == 1)
    def _():
        @pl.when(f == 0)
        def _():
            lo, hi = _unpack_bf16(x_ref[...])
            xb_ref[...] = jnp.concatenate(
                [lo, hi], axis=1).astype(jnp.bfloat16)

        xb = xb_ref[...]
        h1 = jnp.dot(xb, w1_ref[0].astype(jnp.bfloat16),
                     preferred_element_type=jnp.float32)
        h3 = jnp.dot(xb, w3_ref[0].astype(jnp.bfloat16),
                     preferred_element_type=jnp.float32)
        a = ((h1 * lax.logistic(h1)) * h3).astype(jnp.bfloat16)
        part = jnp.dot(a, w2_ref[0].astype(jnp.bfloat16),
                       preferred_element_type=jnp.float32)

        @pl.when(f == 0)
        def _():
            acc_ref[...] = part

        @pl.when((f > 0) & (f < FT - 1))
        def _():
            acc_ref[...] = acc_ref[...] + part

        @pl.when(f == FT - 1)
        def _():
            out = acc_ref[...] + part
            o_ref[...] = _pack_bf16(out[:, :HW], out[:, HW:])


def _ffn_grouped(blk_e, blk_act, blk_ob, x_pad, w1, w3, w2):
    def feff(b, f, act):
        return jnp.where(act[b] == 1, f, FT - 1)

    return pl.pallas_call(
        _ffn_kernel,
        grid_spec=pltpu.PrefetchScalarGridSpec(
            num_scalar_prefetch=3,
            grid=(NBLK, FT),
            in_specs=[
                pl.BlockSpec(
                    (BLK, HW),
                    lambda b, f, be, act, ob: (
                        jnp.where(act[b] == 1, b, 0), 0)),
                pl.BlockSpec(
                    (1, H, FFT),
                    lambda b, f, be, act, ob: (be[b], 0, feff(b, f, act))),
                pl.BlockSpec(
                    (1, H, FFT),
                    lambda b, f, be, act, ob: (be[b], 0, feff(b, f, act))),
                pl.BlockSpec(
                    (1, FFT, H),
                    lambda b, f, be, act, ob: (be[b], feff(b, f, act), 0)),
            ],
            out_specs=pl.BlockSpec(
                (BLK, HW), lambda b, f, be, act, ob: (ob[b], 0)),
            scratch_shapes=[
                pltpu.VMEM((BLK, H), jnp.bfloat16),
                pltpu.VMEM((BLK, H), jnp.float32),
            ],
        ),
        out_shape=jax.ShapeDtypeStruct((P, HW), jnp.int32),
    )(blk_e, blk_act, blk_ob, x_pad, w1, w3, w2)


def _sc_gather_body(outpad, pos, gat, rows0_v, rows1_v, p0_v, p1_v, semg,
                    semw):
    wid = lax.axis_index("s") * 2 + lax.axis_index("c")
    tbase = wid * TPT
    pltpu.sync_copy(pos.at[0, pl.ds(tbase, CH)], p0_v)
    pltpu.sync_copy(pos.at[1, pl.ds(tbase, CH)], p1_v)
    g0 = pltpu.async_copy(outpad.at[p0_v], rows0_v, semg)
    g1 = pltpu.async_copy(outpad.at[p1_v], rows1_v, semg)
    g0.wait()
    w0 = pltpu.async_copy(rows0_v, gat.at[0, pl.ds(tbase, CH)], semw)
    g1.wait()
    w1 = pltpu.async_copy(rows1_v, gat.at[1, pl.ds(tbase, CH)], semw)
    w0.wait()
    w1.wait()


def _gather_sc(out_pad, pos_slots):
    mesh = plsc.VectorSubcoreMesh(core_axis_name="c", subcore_axis_name="s")
    f = functools.partial(
        pl.kernel,
        mesh=mesh,
        out_type=jax.ShapeDtypeStruct((TOPK, T, HW), jnp.int32),
        scratch_types=[
            pltpu.VMEM((CH, HW), jnp.int32),
            pltpu.VMEM((CH, HW), jnp.int32),
            pltpu.VMEM((CH,), jnp.int32),
            pltpu.VMEM((CH,), jnp.int32),
            pltpu.SemaphoreType.DMA,
            pltpu.SemaphoreType.DMA,
        ],
    )(_sc_gather_body)
    return f(out_pad, pos_slots)


def _combine_kernel(inp_ref, g_ref, tw_ref, o_ref):
    tw0 = tw_ref[:, 0:1]
    tw1 = tw_ref[:, 1:2]
    lo0, hi0 = _unpack_bf16(g_ref[0])
    lo1, hi1 = _unpack_bf16(g_ref[1])
    g0 = jnp.concatenate([lo0, hi0], axis=1)
    g1 = jnp.concatenate([lo1, hi1], axis=1)
    o_ref[...] = inp_ref[...] + tw0 * g0 + tw1 * g1


def _combine(inp, gathered, tw):
    nb = 8
    tb = T // nb
    return pl.pallas_call(
        _combine_kernel,
        grid=(nb,),
        in_specs=[
            pl.BlockSpec((tb, H), lambda i: (i, 0)),
            pl.BlockSpec((TOPK, tb, HW), lambda i: (0, i, 0)),
            pl.BlockSpec((tb, LANES), lambda i: (i, 0)),
        ],
        out_specs=pl.BlockSpec((tb, H), lambda i: (i, 0)),
        out_shape=jax.ShapeDtypeStruct((T, H), jnp.float32),
    )(inp, gathered, tw)


def kernel(hidden_states, rms_weight, gate_w, w1, w2, w3):
    b, s, h = hidden_states.shape
    hs = hidden_states.reshape(T, H)
    rmsw = rms_weight.reshape(1, H)
    gwp = jnp.pad(gate_w, ((0, 0), (0, LANES - E)))

    hsn, pos128, tw128, blk = _routing(hs, rmsw, gwp)
    pos_slots = jnp.transpose(pos128[:, :TOPK]).astype(jnp.int32)
    blk_e = blk[0, :NBLK]
    act = blk[1, :NBLK]
    blk_ob = blk[2, :NBLK]

    x_pad = _scatter_sc(hsn, pos_slots)
    out_pad = _ffn_grouped(blk_e, act, blk_ob, x_pad, w1, w3, w2)
    gathered = _gather_sc(out_pad, pos_slots)
    final = _combine(hs, gathered, tw128)
    return final.reshape(b, s, h)
